# baseline (speedup 1.0000x reference)
from functools import partial

import jax
import jax.numpy as jnp
from jax.experimental import pallas as pl
from jax.experimental.pallas import tpu as pltpu

_N_COMS = 8


def _phi_kernel(z_ref, s_ref, phi_ref):
    phi_ref[...] = jnp.dot(z_ref[...], s_ref[...],
                           preferred_element_type=jnp.float32)


def _fused_kernel(phi_ref, x_ref, w_ref, mask_ref, xp_ref,
                  w_scr, m_scr, x_scr, slab_scr, xsem, sems,
                  *, n_coms: int, n_outer: int, n_inner: int, n_half: int):
    o = pl.program_id(0)
    i = pl.program_id(1)

    @pl.when(i == 0)
    def _():
        pltpu.make_async_copy(x_ref, x_scr, xsem).start()
        phi = phi_ref[...]
        phi = phi - jnp.max(phi, axis=0, keepdims=True)
        e = jnp.exp(phi)
        p = e / jnp.sum(e, axis=0, keepdims=True)
        r = jnp.sum(p, axis=1, keepdims=True)
        w = p * (float(n_coms) - r)
        w_scr[...] = w
        m_scr[...] = (w == jnp.max(w, axis=1, keepdims=True)).astype(jnp.float32)
        row = pl.ds(o * n_half, n_half)
        w_ref[...] = w_scr[row, :]
        mask_ref[...] = m_scr[row, :].astype(jnp.int32)
        pltpu.make_async_copy(x_ref, x_scr, xsem).wait()

    for oo in range(n_outer):
        for ii in range(n_inner):
            @pl.when((o == oo) & (i == ii))
            def _(oo=oo, ii=ii):
                c = oo * n_inner + ii
                buf = ii % 2
                if ii >= 2:
                    pltpu.make_async_copy(slab_scr.at[buf], xp_ref.at[c - 2],
                                          sems.at[buf]).wait()
                slab_scr[buf] = x_scr[...] * m_scr[:, c:c + 1]
                pltpu.make_async_copy(slab_scr.at[buf], xp_ref.at[c],
                                      sems.at[buf]).start()
                if ii == n_inner - 1:
                    if n_inner >= 2:
                        pltpu.make_async_copy(slab_scr.at[1 - buf],
                                              xp_ref.at[c - 1],
                                              sems.at[1 - buf]).wait()
                    pltpu.make_async_copy(slab_scr.at[buf], xp_ref.at[c],
                                          sems.at[buf]).wait()


def kernel(x, z):
    N, D = x.shape
    Nz, F = z.shape
    assert Nz == N
    C = _N_COMS
    per = F // C

    tn = 1024 if N > 1024 else N
    n_tiles = pl.cdiv(N, tn)
    tz = 1024 if N > 1024 else N
    nz_tiles = pl.cdiv(N, tz)

    S = (jnp.equal(jnp.arange(F)[:, None] // per,
                   jnp.arange(C)[None, :]).astype(z.dtype)) * (1.0 / per)

    n_outer = 2 if n_tiles % 2 == 0 else 1
    n_inner = n_tiles // n_outer

    nz_outer = 2 if nz_tiles % 2 == 0 else 1
    nz_inner = nz_tiles // nz_outer
    phi = pl.pallas_call(
        _phi_kernel,
        out_shape=jax.ShapeDtypeStruct((N, C), jnp.float32),
        grid=(nz_outer, nz_inner),
        in_specs=[
            pl.BlockSpec((tz, F), lambda o, i: (o * nz_inner + i, 0)),
            pl.BlockSpec((F, C), lambda o, i: (0, 0)),
        ],
        out_specs=pl.BlockSpec((tz, C), lambda o, i: (o * nz_inner + i, 0)),
        compiler_params=pltpu.CompilerParams(
            dimension_semantics=("parallel", "arbitrary"),
            vmem_limit_bytes=64 * 1024 * 1024),
    )(z, S)

    nc_outer = 2 if C % 2 == 0 and N % 16 == 0 else 1
    nc_inner = C // nc_outer
    n_half = N // nc_outer

    node_weight, node_mask, x_parts = pl.pallas_call(
        partial(_fused_kernel, n_coms=C, n_outer=nc_outer,
                n_inner=nc_inner, n_half=n_half),
        out_shape=(jax.ShapeDtypeStruct((N, C), jnp.float32),
                   jax.ShapeDtypeStruct((N, C), jnp.int32),
                   jax.ShapeDtypeStruct((C, N, D), x.dtype)),
        grid=(nc_outer, nc_inner),
        in_specs=[
            pl.BlockSpec((N, C), lambda o, i: (0, 0)),
            pl.BlockSpec(memory_space=pl.ANY),
        ],
        out_specs=(pl.BlockSpec((n_half, C), lambda o, i: (o, 0)),
                   pl.BlockSpec((n_half, C), lambda o, i: (o, 0)),
                   pl.BlockSpec(memory_space=pl.ANY)),
        scratch_shapes=[pltpu.VMEM((N, C), jnp.float32),
                        pltpu.VMEM((N, C), jnp.float32),
                        pltpu.VMEM((N, D), x.dtype),
                        pltpu.VMEM((2, N, D), x.dtype),
                        pltpu.SemaphoreType.DMA,
                        pltpu.SemaphoreType.DMA((2,))],
        compiler_params=pltpu.CompilerParams(
            dimension_semantics=("parallel", "arbitrary"),
            vmem_limit_bytes=64 * 1024 * 1024),
    )(phi, x)

    return node_weight, node_mask, x_parts


# speedup vs baseline: 1.0647x; 1.0647x over previous
from functools import partial

import jax
import jax.numpy as jnp
from jax.experimental import pallas as pl
from jax.experimental.pallas import tpu as pltpu

_N_COMS = 8


def _phi_kernel(z_ref, s_ref, phi_ref):
    phi_ref[...] = jnp.dot(z_ref[...], s_ref[...],
                           preferred_element_type=jnp.float32)


def _fused_kernel(phi_ref, x_ref, w_ref, mask_ref, xp_ref, w_scr, m_scr,
                  *, n_coms: int, tn: int, n_inner: int, n_half: int):
    o = pl.program_id(0)
    i = pl.program_id(1)

    @pl.when(i == 0)
    def _():
        phi = phi_ref[...]
        phi = phi - jnp.max(phi, axis=0, keepdims=True)
        e = jnp.exp(phi)
        p = e / jnp.sum(e, axis=0, keepdims=True)
        r = jnp.sum(p, axis=1, keepdims=True)
        w = p * (float(n_coms) - r)
        w_scr[...] = w
        m_scr[...] = (w == jnp.max(w, axis=1, keepdims=True)).astype(jnp.float32)
        row = pl.ds(o * n_half, n_half)
        w_ref[...] = w_scr[row, :]
        mask_ref[...] = m_scr[row, :].astype(jnp.int32)

    t = o * n_inner + i
    row = pl.ds(t * tn, tn)
    x = x_ref[row, :]
    for c in range(n_coms):
        xp_ref[c] = x * m_scr[row, c:c + 1]


def kernel(x, z):
    N, D = x.shape
    Nz, F = z.shape
    assert Nz == N
    C = _N_COMS
    per = F // C

    tn = 1024 if N > 1024 else N
    n_tiles = pl.cdiv(N, tn)
    tz = 1024 if N > 1024 else N
    nz_tiles = pl.cdiv(N, tz)

    S = (jnp.equal(jnp.arange(F)[:, None] // per,
                   jnp.arange(C)[None, :]).astype(z.dtype)) * (1.0 / per)

    n_outer = 2 if n_tiles % 2 == 0 else 1
    n_inner = n_tiles // n_outer

    nz_outer = 2 if nz_tiles % 2 == 0 else 1
    nz_inner = nz_tiles // nz_outer
    phi = pl.pallas_call(
        _phi_kernel,
        out_shape=jax.ShapeDtypeStruct((N, C), jnp.float32),
        grid=(nz_outer, nz_inner),
        in_specs=[
            pl.BlockSpec((tz, F), lambda o, i: (o * nz_inner + i, 0)),
            pl.BlockSpec((F, C), lambda o, i: (0, 0)),
        ],
        out_specs=pl.BlockSpec((tz, C), lambda o, i: (o * nz_inner + i, 0)),
        compiler_params=pltpu.CompilerParams(
            dimension_semantics=("parallel", "arbitrary"),
            vmem_limit_bytes=64 * 1024 * 1024),
    )(z, S)

    n_half = N // n_outer

    node_weight, node_mask, x_parts = pl.pallas_call(
        partial(_fused_kernel, n_coms=C, tn=tn, n_inner=n_inner,
                n_half=n_half),
        out_shape=(jax.ShapeDtypeStruct((N, C), jnp.float32),
                   jax.ShapeDtypeStruct((N, C), jnp.int32),
                   jax.ShapeDtypeStruct((C, N, D), x.dtype)),
        grid=(n_outer, n_inner),
        in_specs=[
            pl.BlockSpec((N, C), lambda o, i: (0, 0)),
            pl.BlockSpec((N, D), lambda o, i: (0, 0)),
        ],
        out_specs=(pl.BlockSpec((n_half, C), lambda o, i: (o, 0)),
                   pl.BlockSpec((n_half, C), lambda o, i: (o, 0)),
                   pl.BlockSpec((C, tn, D),
                                lambda o, i: (0, o * n_inner + i, 0))),
        scratch_shapes=[pltpu.VMEM((N, C), jnp.float32),
                        pltpu.VMEM((N, C), jnp.float32)],
        compiler_params=pltpu.CompilerParams(
            dimension_semantics=("parallel", "arbitrary"),
            vmem_limit_bytes=64 * 1024 * 1024),
    )(phi, x)

    return node_weight, node_mask, x_parts


# speedup vs baseline: 1.0655x; 1.0008x over previous
from functools import partial

import jax
import jax.numpy as jnp
from jax.experimental import pallas as pl
from jax.experimental.pallas import tpu as pltpu

_N_COMS = 8


def _phi_kernel(z_ref, s_ref, phi_ref):
    phi_ref[...] = jnp.dot(z_ref[...], s_ref[...],
                           preferred_element_type=jnp.float32)


def _fused_kernel(phi_ref, x_ref, w_ref, mask_ref, xp_ref, w_scr, m_scr,
                  *, n_coms: int, tn: int, n_inner: int):
    o = pl.program_id(0)
    i = pl.program_id(1)

    @pl.when(i == 0)
    def _():
        phi = phi_ref[...]
        phi = phi - jnp.max(phi, axis=0, keepdims=True)
        e = jnp.exp(phi)
        p = e / jnp.sum(e, axis=0, keepdims=True)
        r = jnp.sum(p, axis=1, keepdims=True)
        w = p * (float(n_coms) - r)
        w_scr[...] = w
        m_scr[...] = (w == jnp.max(w, axis=1, keepdims=True)).astype(jnp.float32)

    t = o * n_inner + i
    w_tile = w_scr[pl.ds(t * tn, tn), :]
    m_tile = m_scr[pl.ds(t * tn, tn), :]
    w_ref[...] = w_tile
    mask_ref[...] = m_tile.astype(jnp.int32)
    x = x_ref[...]
    for c in range(n_coms):
        xp_ref[c] = x * m_tile[:, c:c + 1]


def kernel(x, z):
    N, D = x.shape
    Nz, F = z.shape
    assert Nz == N
    C = _N_COMS
    per = F // C

    tn = 1024 if N > 1024 else N
    n_tiles = pl.cdiv(N, tn)
    tz = 1024 if N > 1024 else N
    nz_tiles = pl.cdiv(N, tz)

    S = (jnp.equal(jnp.arange(F)[:, None] // per,
                   jnp.arange(C)[None, :]).astype(z.dtype)) * (1.0 / per)

    n_outer = 2 if n_tiles % 2 == 0 else 1
    n_inner = n_tiles // n_outer

    nz_outer = 2 if nz_tiles % 2 == 0 else 1
    nz_inner = nz_tiles // nz_outer
    phi = pl.pallas_call(
        _phi_kernel,
        out_shape=jax.ShapeDtypeStruct((N, C), jnp.float32),
        grid=(nz_outer, nz_inner),
        in_specs=[
            pl.BlockSpec((tz, F), lambda o, i: (o * nz_inner + i, 0)),
            pl.BlockSpec((F, C), lambda o, i: (0, 0)),
        ],
        out_specs=pl.BlockSpec((tz, C), lambda o, i: (o * nz_inner + i, 0)),
        compiler_params=pltpu.CompilerParams(
            dimension_semantics=("parallel", "arbitrary"),
            vmem_limit_bytes=64 * 1024 * 1024),
    )(z, S)

    node_weight, node_mask, x_parts = pl.pallas_call(
        partial(_fused_kernel, n_coms=C, tn=tn, n_inner=n_inner),
        out_shape=(jax.ShapeDtypeStruct((N, C), jnp.float32),
                   jax.ShapeDtypeStruct((N, C), jnp.int32),
                   jax.ShapeDtypeStruct((C, N, D), x.dtype)),
        grid=(n_outer, n_inner),
        in_specs=[
            pl.BlockSpec((N, C), lambda o, i: (0, 0)),
            pl.BlockSpec((tn, D), lambda o, i: (o * n_inner + i, 0)),
        ],
        out_specs=(pl.BlockSpec((tn, C), lambda o, i: (o * n_inner + i, 0)),
                   pl.BlockSpec((tn, C), lambda o, i: (o * n_inner + i, 0)),
                   pl.BlockSpec((C, tn, D), lambda o, i: (0, o * n_inner + i, 0))),
        scratch_shapes=[pltpu.VMEM((N, C), jnp.float32),
                        pltpu.VMEM((N, C), jnp.float32)],
        compiler_params=pltpu.CompilerParams(
            dimension_semantics=("parallel", "arbitrary"),
            vmem_limit_bytes=64 * 1024 * 1024),
    )(phi, x)

    return node_weight, node_mask, x_parts
